# Initial kernel scaffold; baseline (speedup 1.0000x reference)
import jax
import jax.numpy as jnp
from jax.experimental import pallas as pl


def kernel(x, up_attr, boundary_attr, Wmu, bmu, Wu1, bu1, gu1, beu1, Wu2, bu2, gu2, beu2, Wb1, bb1, gb1, beb1, Wb2, bb2, gb2, beb2, Wc, bc, gc, bec, up_index, boundary_index):
    raise NotImplementedError("write your pallas kernel here")


# speedup vs baseline: 1.9480x; 1.9480x over previous
import functools

import jax
import jax.numpy as jnp
from jax import lax
from jax.experimental import pallas as pl
from jax.experimental.pallas import tpu as pltpu
from jax.experimental.pallas import tpu_sc as plsc

_NC = 2
_NS = 16
_CHUNK = 128


def _stage1_body(up_ref, x_ref, wtop_ref, wbot_ref, bmu_ref, uw_ref, xw_ref):
    uw_ref[...] = (
        jnp.dot(up_ref[...], wbot_ref[...], preferred_element_type=jnp.float32)
        + bmu_ref[...]
    )

    @pl.when(pl.program_id(0) == 0)
    def _():
        xw_ref[...] = jnp.dot(
            x_ref[...], wtop_ref[...], preferred_element_type=jnp.float32
        )


def _stage1(up_attr, x, wtop, wbot, bmu2):
    E, D = up_attr.shape
    N = x.shape[0]
    BE = 8000
    grid = E // BE
    return pl.pallas_call(
        _stage1_body,
        grid=(grid,),
        in_specs=[
            pl.BlockSpec((BE, D), lambda i: (i, 0)),
            pl.BlockSpec((N, D), lambda i: (0, 0)),
            pl.BlockSpec((D, D), lambda i: (0, 0)),
            pl.BlockSpec((D, D), lambda i: (0, 0)),
            pl.BlockSpec((1, D), lambda i: (0, 0)),
        ],
        out_specs=[
            pl.BlockSpec((BE, D), lambda i: (i, 0)),
            pl.BlockSpec((N, D), lambda i: (0, 0)),
        ],
        out_shape=[
            jax.ShapeDtypeStruct((E, D), jnp.float32),
            jax.ShapeDtypeStruct((N, D), jnp.float32),
        ],
    )(up_attr, x, wtop, wbot, bmu2)


def _sc_segment_body(NP, E, EBP, D,
                     xw_hbm, uw_hbm, battr_hbm, src_hbm, dst_hbm,
                     bsrc_hbm, bdst_hbm,
                     up_parts_hbm, b_parts_hbm,
                     idx_s, idx_d, uw_v, xg_v, acc, sem):
    c = lax.axis_index("c")
    s = lax.axis_index("s")
    gw = s * _NC + c
    nw = _NC * _NS
    rows_per_sub = NP // _NS
    vecs = _CHUNK * D // 16

    def _fill_zero(i, _):
        r = i // (D // 16)
        l = (i % (D // 16)) * 16
        xg_v[r, pl.ds(l, 16)] = jnp.zeros((16,), jnp.float32)
        return 0

    def _zero_acc():
        base = s * rows_per_sub
        nfull = rows_per_sub // _CHUNK
        rem = rows_per_sub - nfull * _CHUNK

        def _z(i, _):
            pltpu.sync_copy(xg_v, acc.at[pl.ds(base + i * _CHUNK, _CHUNK)])
            return 0

        lax.fori_loop(0, nfull, _z, 0)
        if rem:
            pltpu.sync_copy(
                xg_v.at[pl.ds(0, rem)],
                acc.at[pl.ds(base + nfull * _CHUNK, rem)],
            )

    lax.fori_loop(0, vecs, _fill_zero, 0)
    _zero_acc()
    plsc.subcore_barrier()

    nchunks = E // _CHUNK

    def _relu_add(i, _):
        r = i // (D // 16)
        l = (i % (D // 16)) * 16
        uw_v[r, pl.ds(l, 16)] = jnp.maximum(
            uw_v[r, pl.ds(l, 16)] + xg_v[r, pl.ds(l, 16)], 0.0
        )
        return 0

    def _up_chunk(k, _):
        base = (gw + k * nw) * _CHUNK
        pltpu.sync_copy(src_hbm.at[pl.ds(base, _CHUNK)], idx_s)
        pltpu.sync_copy(dst_hbm.at[pl.ds(base, _CHUNK)], idx_d)
        pltpu.async_copy(xw_hbm.at[idx_s], xg_v, sem).wait()
        pltpu.sync_copy(uw_hbm.at[pl.ds(base, _CHUNK)], uw_v)
        lax.fori_loop(0, vecs, _relu_add, 0)
        pltpu.sync_copy(uw_v, acc.at[idx_d], add=True)
        return 0

    my_chunks = (nchunks - gw + nw - 1) // nw
    lax.fori_loop(0, my_chunks, _up_chunk, 0)
    plsc.subcore_barrier()

    row0 = s * rows_per_sub
    pltpu.sync_copy(
        acc.at[pl.ds(row0, rows_per_sub)],
        up_parts_hbm.at[c, pl.ds(row0, rows_per_sub)],
    )
    plsc.subcore_barrier()

    lax.fori_loop(0, vecs, _fill_zero, 0)
    _zero_acc()
    plsc.subcore_barrier()

    nbchunks = EBP // _CHUNK

    def _b_chunk(k, _):
        base = (gw + k * nw) * _CHUNK
        pltpu.sync_copy(bsrc_hbm.at[pl.ds(base, _CHUNK)], idx_s)
        pltpu.sync_copy(bdst_hbm.at[pl.ds(base, _CHUNK)], idx_d)
        pltpu.async_copy(battr_hbm.at[idx_s], xg_v, sem).wait()
        pltpu.sync_copy(xg_v, acc.at[idx_d], add=True)
        return 0

    my_bchunks = (nbchunks - gw + nw - 1) // nw
    lax.fori_loop(0, my_bchunks, _b_chunk, 0)
    plsc.subcore_barrier()

    pltpu.sync_copy(
        acc.at[pl.ds(row0, rows_per_sub)],
        b_parts_hbm.at[c, pl.ds(row0, rows_per_sub)],
    )


def _sc_segment(xw, uw, battr, src, dst, bsrc, bdst, NP):
    E = src.shape[0]
    EBP = bsrc.shape[0]
    D = xw.shape[1]
    mesh = plsc.VectorSubcoreMesh(
        core_axis_name="c", subcore_axis_name="s",
        num_cores=_NC, num_subcores=_NS,
    )
    fn = pl.kernel(
        functools.partial(_sc_segment_body, NP, E, EBP, D),
        out_type=[
            jax.ShapeDtypeStruct((_NC, NP, D), jnp.float32),
            jax.ShapeDtypeStruct((_NC, NP, D), jnp.float32),
        ],
        mesh=mesh,
        scratch_types=[
            pltpu.VMEM((_CHUNK,), jnp.int32),
            pltpu.VMEM((_CHUNK,), jnp.int32),
            pltpu.VMEM((_CHUNK, D), jnp.float32),
            pltpu.VMEM((_CHUNK, D), jnp.float32),
            pltpu.VMEM_SHARED((NP, D), jnp.float32),
            pltpu.SemaphoreType.DMA,
        ],
    )
    return fn(xw, uw, battr, src, dst, bsrc, bdst)


def _stage2_body(N,
                 up_parts, b_parts, x_ref,
                 wu1, bu1, gu1, beu1, wu2, bu2, gu2, beu2,
                 wb1, bb1, gb1, beb1, wb2, bb2, gb2, beb2,
                 wc1, wc2, bc, gc, bec, out_ref):
    def bn_relu(t, g, b):
        mu = jnp.mean(t, axis=0, keepdims=True)
        var = jnp.mean((t - mu) ** 2, axis=0, keepdims=True)
        return jnp.maximum(g * (t - mu) / jnp.sqrt(var + 1e-5) + b, 0.0)

    xv = x_ref[...]
    ou = up_parts[0, :N, :] + up_parts[1, :N, :] + xv
    ob = b_parts[0, :N, :] + b_parts[1, :N, :] + xv

    h1 = bn_relu(jnp.dot(ou, wu1[...], preferred_element_type=jnp.float32)
                 + bu1[...], gu1[...], beu1[...])
    h1 = bn_relu(jnp.dot(h1, wu2[...], preferred_element_type=jnp.float32)
                 + bu2[...], gu2[...], beu2[...])
    h2 = bn_relu(jnp.dot(ob, wb1[...], preferred_element_type=jnp.float32)
                 + bb1[...], gb1[...], beb1[...])
    h2 = bn_relu(jnp.dot(h2, wb2[...], preferred_element_type=jnp.float32)
                 + bb2[...], gb2[...], beb2[...])
    comb = (jnp.dot(h1, wc1[...], preferred_element_type=jnp.float32)
            + jnp.dot(h2, wc2[...], preferred_element_type=jnp.float32)
            + bc[...])
    out_ref[...] = bn_relu(comb, gc[...], bec[...])


def _stage2(up_parts, b_parts, x, *weights):
    N, D = x.shape
    H = weights[0].shape[1]
    return pl.pallas_call(
        functools.partial(_stage2_body, N),
        out_shape=jax.ShapeDtypeStruct((N, H), jnp.float32),
    )(up_parts, b_parts, x, *weights)


def kernel(x, up_attr, boundary_attr, Wmu, bmu, Wu1, bu1, gu1, beu1, Wu2, bu2,
           gu2, beu2, Wb1, bb1, gb1, beb1, Wb2, bb2, gb2, beb2, Wc, bc, gc,
           bec, up_index, boundary_index):
    N, D = x.shape
    E = up_index.shape[1]
    EB = boundary_index.shape[1]
    H = Wu1.shape[1]

    NP = ((N + 1 + _NS * 8 - 1) // (_NS * 8)) * (_NS * 8)

    wtop = Wmu[:D]
    wbot = Wmu[D:]
    bmu2 = bmu.reshape(1, D)

    uw, xw = _stage1(up_attr, x, wtop, wbot, bmu2)

    src = up_index[0]
    dst = up_index[1]
    EBP = ((EB + _CHUNK - 1) // _CHUNK) * _CHUNK
    pad = EBP - EB
    bsrc = jnp.concatenate([boundary_index[0],
                            jnp.zeros((pad,), jnp.int32)]) if pad else boundary_index[0]
    bdst = jnp.concatenate([boundary_index[1],
                            jnp.full((pad,), N, jnp.int32)]) if pad else boundary_index[1]

    up_parts, b_parts = _sc_segment(xw, uw, boundary_attr, src, dst, bsrc,
                                    bdst, NP)

    out = _stage2(
        up_parts, b_parts, x,
        Wu1, bu1.reshape(1, H), gu1.reshape(1, H), beu1.reshape(1, H),
        Wu2, bu2.reshape(1, H), gu2.reshape(1, H), beu2.reshape(1, H),
        Wb1, bb1.reshape(1, H), gb1.reshape(1, H), beb1.reshape(1, H),
        Wb2, bb2.reshape(1, H), gb2.reshape(1, H), beb2.reshape(1, H),
        Wc[:H], Wc[H:], bc.reshape(1, H), gc.reshape(1, H), bec.reshape(1, H),
    )
    return out
